# baseline (speedup 1.0000x reference)
import functools

import jax
import jax.numpy as jnp
import numpy as np
from jax import lax
from jax.experimental import pallas as pl
from jax.experimental.pallas import tpu as pltpu
from jax.experimental.pallas import tpu_sc as plsc

N = 10000
E = 320000
H = 128
NF = 128
EC = 128
L = 3
CUTOFF = 10.0

NC = 2
NS = 16
NW = NC * NS
EW = E // NW
CH = 40
NCH = EW // CH
ECH = E // CH
NP = 5
PCH = NCH // NP
NPAD = 10240
RPT = NPAD // NS
ICH = 80
ZCH = NPAD // NW // ICH

_MESH = dict(core_axis_name="c", subcore_axis_name="s", num_cores=NC,
             num_subcores=NS)


def _filter_body(ea_ref, el_ref, emb_ref, fw1_ref, fb1_ref, fw2_ref, fb2_ref,
                 cw1t0_ref, wf0_ref, wf1_ref, wf2_ref, xemb_ref):
  BE = ea_ref.shape[0]
  a = ea_ref[...].astype(jnp.bfloat16)
  c = (el_ref[...] <= CUTOFF).astype(jnp.float32)
  outs = (wf0_ref, wf1_ref, wf2_ref)
  for i in range(L):
    t = jnp.dot(a, fw1_ref[i].astype(jnp.bfloat16),
                preferred_element_type=jnp.float32)
    t = jax.nn.gelu(t + fb1_ref[i]).astype(jnp.bfloat16)
    t = jnp.dot(t, fw2_ref[i].astype(jnp.bfloat16),
                preferred_element_type=jnp.float32)
    w = (t + fb2_ref[i]) * c
    outs[i][...] = w.reshape(BE // CH, CH, NF)

  @pl.when(pl.program_id(0) == 0)
  def _():
    xemb_ref[...] = jnp.dot(emb_ref[...], cw1t0_ref[...],
                            preferred_element_type=jnp.float32)


def _filters(edge_attr, edge_length, emb_table, f_w1t, f_b1, f_w2t, f_b2,
             c_w1t0):
  BE = 1280
  grid = E // BE
  full = lambda shape: pl.BlockSpec(shape, lambda b: tuple(0 for _ in shape))
  wf_spec = pl.BlockSpec((BE // CH, CH, NF), lambda b: (b, 0, 0))
  wf_shape = jax.ShapeDtypeStruct((ECH, CH, NF), jnp.float32)
  return pl.pallas_call(
      _filter_body,
      grid=(grid,),
      in_specs=[
          pl.BlockSpec((BE, EC), lambda b: (b, 0)),
          pl.BlockSpec((BE, 1), lambda b: (b, 0)),
          full((100, H)),
          full((L, EC, NF)),
          full((L, NF)),
          full((L, NF, NF)),
          full((L, NF)),
          full((H, NF)),
      ],
      out_specs=[wf_spec, wf_spec, wf_spec, full((100, NF))],
      out_shape=[wf_shape, wf_shape, wf_shape,
                 jax.ShapeDtypeStruct((100, NF), jnp.float32)],
  )(edge_attr, edge_length.reshape(E, 1), emb_table, f_w1t, f_b1, f_w2t,
    f_b2, c_w1t0)


def _init_gather_body(z_hbm, emb_hbm, xemb_hbm, h0_hbm, xl0_hbm,
                      zslab, hrows, xrows, sem):
  wid = lax.axis_index("c") * NS + lax.axis_index("s")
  pltpu.sync_copy(z_hbm.at[wid], zslab)

  @pl.loop(0, ZCH)
  def _(j):
    pltpu.async_copy(emb_hbm.at[zslab.at[j]], hrows, sem).wait()
    pltpu.async_copy(xemb_hbm.at[zslab.at[j]], xrows, sem).wait()
    base = (wid * ZCH + j) * ICH
    pltpu.sync_copy(hrows, h0_hbm.at[pl.ds(base, ICH)])
    pltpu.sync_copy(xrows, xl0_hbm.at[pl.ds(base, ICH)])


def _init_gather(z_pad, emb_table, xemb):
  return pl.kernel(
      _init_gather_body,
      out_type=[
          jax.ShapeDtypeStruct((NPAD, H), jnp.float32),
          jax.ShapeDtypeStruct((NPAD, NF), jnp.float32),
      ],
      mesh=plsc.VectorSubcoreMesh(**_MESH),
      scratch_types=[
          pltpu.VMEM((ZCH, ICH), jnp.int32),
          pltpu.VMEM((ICH, H), jnp.float32),
          pltpu.VMEM((ICH, NF), jnp.float32),
          pltpu.SemaphoreType.DMA,
      ],
  )(z_pad, emb_table, xemb)


def _mp_body(xl_hbm, wf_hbm, src_hbm, dst_hbm, out_hbm,
             src_sl, dst_sl, rows0, rows1, wfb0, wfb1, agg_sp,
             g0, g1, w0, w1, s0, s1):
  cid = lax.axis_index("c")
  sid = lax.axis_index("s")
  wid = cid * NS + sid
  rows = (rows0, rows1)
  wfb = (wfb0, wfb1)
  gsem = (g0, g1)
  wsem = (w0, w1)
  ssem = (s0, s1)

  @pl.loop(0, CH * (NF // 16))
  def _(t):
    r = t // (NF // 16)
    v = t % (NF // 16)
    rows0[r, pl.ds(v * 16, 16)] = jnp.zeros((16,), jnp.float32)

  @pl.loop(0, RPT // CH)
  def _(k):
    pltpu.sync_copy(rows0, agg_sp.at[pl.ds(sid * RPT + k * CH, CH)])

  plsc.subcore_barrier()

  @pl.loop(0, NP)
  def _(h):
    pltpu.sync_copy(src_hbm.at[wid, h], src_sl)
    pltpu.sync_copy(dst_hbm.at[wid, h], dst_sl)
    cbase = (wid * NP + h) * PCH

    def issue(k, b):
      pltpu.async_copy(xl_hbm.at[src_sl.at[k]], rows[b], gsem[b])
      pltpu.async_copy(wf_hbm.at[cbase + k], wfb[b], wsem[b])

    def phase(k, b):
      @pl.when(k > 0)
      def _():
        pltpu.make_async_copy(rows[1 - b], agg_sp.at[dst_sl.at[k - 1]],
                              ssem[1 - b]).wait()

      @pl.when(k + 1 < PCH)
      def _():
        issue(k + 1, 1 - b)

      pltpu.make_async_copy(xl_hbm.at[src_sl.at[k]], rows[b], gsem[b]).wait()
      pltpu.make_async_copy(wf_hbm.at[cbase + k], wfb[b], wsem[b]).wait()

      for e in range(CH):
        for v in range(NF // 16):
          sl = pl.ds(v * 16, 16)
          rows[b][e, sl] = rows[b][e, sl] * wfb[b][e, sl]

      pltpu.async_copy(rows[b], agg_sp.at[dst_sl.at[k]], ssem[b], add=True)

    issue(0, 0)

    @pl.loop(0, PCH // 2)
    def _(m):
      phase(2 * m, 0)
      phase(2 * m + 1, 1)

    pltpu.make_async_copy(rows[1], agg_sp.at[dst_sl.at[PCH - 1]],
                          ssem[1]).wait()

  plsc.subcore_barrier()
  pltpu.sync_copy(agg_sp.at[pl.ds(sid * RPT, RPT)],
                  out_hbm.at[cid, pl.ds(sid * RPT, RPT)])


def _message_pass(xl, wf, src_r, dst_r):
  return pl.kernel(
      _mp_body,
      out_type=jax.ShapeDtypeStruct((NC, NPAD, NF), jnp.float32),
      mesh=plsc.VectorSubcoreMesh(**_MESH),
      scratch_types=[
          pltpu.VMEM((PCH, CH), jnp.int32),
          pltpu.VMEM((PCH, CH), jnp.int32),
          pltpu.VMEM((CH, NF), jnp.float32),
          pltpu.VMEM((CH, NF), jnp.float32),
          pltpu.VMEM((CH, NF), jnp.float32),
          pltpu.VMEM((CH, NF), jnp.float32),
          pltpu.VMEM_SHARED((NPAD, NF), jnp.float32),
          pltpu.SemaphoreType.DMA,
          pltpu.SemaphoreType.DMA,
          pltpu.SemaphoreType.DMA,
          pltpu.SemaphoreType.DMA,
          pltpu.SemaphoreType.DMA,
          pltpu.SemaphoreType.DMA,
      ],
  )(xl, wf, src_r, dst_r)


def _node_body(has_next, h_ref, agga_ref, aggb_ref, cw2t_ref, cb2_ref,
               lwt_ref, lb_ref, cw1t_ref, hn_ref, xl_ref):
  agg = agga_ref[0] + aggb_ref[0]
  x2 = jnp.dot(agg, cw2t_ref[...], preferred_element_type=jnp.float32)
  x2 = jax.nn.gelu(x2 + cb2_ref[...])
  x2 = jnp.dot(x2, lwt_ref[...], preferred_element_type=jnp.float32)
  hn = h_ref[...] + x2 + lb_ref[...]
  hn_ref[...] = hn
  if has_next:
    xl_ref[...] = jnp.dot(hn, cw1t_ref[...],
                          preferred_element_type=jnp.float32)


def _node_update(h, agg2, c_w2t, c_b2, l_wt, l_b, c_w1t_next):
  BN = 1000
  grid = N // BN
  has_next = c_w1t_next is not None
  if not has_next:
    c_w1t_next = c_w2t
  full = lambda shape: pl.BlockSpec(shape, lambda b: tuple(0 for _ in shape))
  return pl.pallas_call(
      functools.partial(_node_body, has_next),
      grid=(grid,),
      in_specs=[
          pl.BlockSpec((BN, H), lambda b: (b, 0)),
          pl.BlockSpec((1, BN, NF), lambda b: (0, b, 0)),
          pl.BlockSpec((1, BN, NF), lambda b: (1, b, 0)),
          full((NF, H)),
          full((H,)),
          full((H, H)),
          full((H,)),
          full((H, NF)),
      ],
      out_specs=[
          pl.BlockSpec((BN, H), lambda b: (b, 0)),
          pl.BlockSpec((BN, NF), lambda b: (b, 0)),
      ],
      out_shape=[
          jax.ShapeDtypeStruct((N, H), jnp.float32),
          jax.ShapeDtypeStruct((N, NF), jnp.float32),
      ],
  )(h, agg2, agg2, c_w2t, c_b2, l_wt, l_b, c_w1t_next)


def kernel(z, edge_index, edge_length, edge_attr, emb_table,
           f_w1, f_b1, f_w2, f_b2, c_w1, c_w2, c_b2, l_w, l_b):
  z = z.astype(jnp.int32)
  src = edge_index[0].astype(jnp.int32).reshape(NW, NP, PCH, CH)
  dst = edge_index[1].astype(jnp.int32).reshape(NW, NP, PCH, CH)
  z_pad = jnp.concatenate([z, jnp.zeros((NPAD - N,), jnp.int32)])
  z_pad = z_pad.reshape(NW, ZCH, ICH)

  f_w1t = f_w1.transpose(0, 2, 1)
  f_w2t = f_w2.transpose(0, 2, 1)
  c_w1t = c_w1.transpose(0, 2, 1)
  c_w2t = c_w2.transpose(0, 2, 1)
  l_wt = l_w.transpose(0, 2, 1)

  wf0, wf1, wf2, xemb = _filters(edge_attr, edge_length, emb_table,
                                 f_w1t, f_b1, f_w2t, f_b2, c_w1t[0])
  wfs = (wf0, wf1, wf2)

  h_pad, xl_pad = _init_gather(z_pad, emb_table, xemb)
  h, xl = h_pad, xl_pad

  for i in range(L):
    agg2 = _message_pass(xl, wfs[i], src, dst)
    nxt = c_w1t[i + 1] if i + 1 < L else None
    h, xl = _node_update(h, agg2, c_w2t[i], c_b2[i], l_wt[i], l_b[i], nxt)
  return h


# speedup vs baseline: 1.0705x; 1.0705x over previous
import functools

import jax
import jax.numpy as jnp
import numpy as np
from jax import lax
from jax.experimental import pallas as pl
from jax.experimental.pallas import tpu as pltpu
from jax.experimental.pallas import tpu_sc as plsc

N = 10000
E = 320000
H = 128
NF = 128
EC = 128
L = 3
CUTOFF = 10.0

NC = 2
NS = 16
NW = NC * NS
EW = E // NW
CH = 40
NCH = EW // CH
ECH = E // CH
NP = 5
PCH = NCH // NP
NPAD = 10240
RPT = NPAD // NS
ICH = 80
ZCH = NPAD // NW // ICH

_MESH = dict(core_axis_name="c", subcore_axis_name="s", num_cores=NC,
             num_subcores=NS)


def _filter_body(ea_ref, emb_ref, fw1_ref, fb1_ref, fw2_ref, fb2_ref,
                 cw1t0_ref, wf0_ref, wf1_ref, wf2_ref, xemb_ref):
  BE = ea_ref.shape[0]
  a = ea_ref[...].astype(jnp.bfloat16)
  outs = (wf0_ref, wf1_ref, wf2_ref)
  for i in range(L):
    t = jnp.dot(a, fw1_ref[i].astype(jnp.bfloat16),
                preferred_element_type=jnp.float32)
    t = jax.nn.gelu(t + fb1_ref[i]).astype(jnp.bfloat16)
    t = jnp.dot(t, fw2_ref[i].astype(jnp.bfloat16),
                preferred_element_type=jnp.float32)
    w = t + fb2_ref[i]
    outs[i][...] = w.reshape(BE // CH, CH, NF)

  @pl.when(pl.program_id(0) == 0)
  def _():
    xemb_ref[...] = jnp.dot(emb_ref[...], cw1t0_ref[...],
                            preferred_element_type=jnp.float32)


def _filters(edge_attr, emb_table, f_w1t, f_b1, f_w2t, f_b2, c_w1t0):
  BE = 1280
  grid = E // BE
  full = lambda shape: pl.BlockSpec(shape, lambda b: tuple(0 for _ in shape))
  wf_spec = pl.BlockSpec((BE // CH, CH, NF), lambda b: (b, 0, 0))
  wf_shape = jax.ShapeDtypeStruct((ECH, CH, NF), jnp.float32)
  return pl.pallas_call(
      _filter_body,
      grid=(grid,),
      in_specs=[
          pl.BlockSpec((BE, EC), lambda b: (b, 0)),
          full((100, H)),
          full((L, EC, NF)),
          full((L, NF)),
          full((L, NF, NF)),
          full((L, NF)),
          full((H, NF)),
      ],
      out_specs=[wf_spec, wf_spec, wf_spec, full((100, NF))],
      out_shape=[wf_shape, wf_shape, wf_shape,
                 jax.ShapeDtypeStruct((100, NF), jnp.float32)],
  )(edge_attr, emb_table, f_w1t, f_b1, f_w2t, f_b2, c_w1t0)


EWV = EW // 16


def _init_gather_body(z_hbm, emb_hbm, xemb_hbm, dst_hbm, el_hbm,
                      h0_hbm, xl0_hbm, dm_hbm,
                      zslab, hrows, xrows, dsl, esl, sem):
  wid = lax.axis_index("c") * NS + lax.axis_index("s")
  pltpu.sync_copy(z_hbm.at[wid], zslab)

  @pl.loop(0, ZCH)
  def _(j):
    pltpu.async_copy(emb_hbm.at[zslab.at[j]], hrows, sem).wait()
    pltpu.async_copy(xemb_hbm.at[zslab.at[j]], xrows, sem).wait()
    base = (wid * ZCH + j) * ICH
    pltpu.sync_copy(hrows, h0_hbm.at[pl.ds(base, ICH)])
    pltpu.sync_copy(xrows, xl0_hbm.at[pl.ds(base, ICH)])

  pltpu.sync_copy(dst_hbm.at[wid, 0], dsl)
  pltpu.sync_copy(el_hbm.at[wid, 0], esl)

  @pl.loop(0, EWV)
  def _(r):
    sl = pl.ds(r * 16, 16)
    dm = jnp.where(esl[sl] <= CUTOFF, dsl[sl], jnp.full((16,), N, jnp.int32))
    dsl[sl] = dm

  pltpu.sync_copy(dsl, dm_hbm.at[wid, 0])


def _init_gather(z_pad, emb_table, xemb, dst_r, el_r):
  return pl.kernel(
      _init_gather_body,
      out_type=[
          jax.ShapeDtypeStruct((NPAD, H), jnp.float32),
          jax.ShapeDtypeStruct((NPAD, NF), jnp.float32),
          jax.ShapeDtypeStruct((NW, 1, EW), jnp.int32),
      ],
      mesh=plsc.VectorSubcoreMesh(**_MESH),
      scratch_types=[
          pltpu.VMEM((ZCH, ICH), jnp.int32),
          pltpu.VMEM((ICH, H), jnp.float32),
          pltpu.VMEM((ICH, NF), jnp.float32),
          pltpu.VMEM((EW,), jnp.int32),
          pltpu.VMEM((EW,), jnp.float32),
          pltpu.SemaphoreType.DMA,
      ],
  )(z_pad, emb_table, xemb, dst_r, el_r)


def _mp_body(xl_hbm, wf_hbm, src_hbm, dst_hbm, out_hbm,
             src_sl, dst_sl, rows0, rows1, wfb0, wfb1, agg_sp,
             g0, g1, w0, w1, s0, s1):
  cid = lax.axis_index("c")
  sid = lax.axis_index("s")
  wid = cid * NS + sid
  rows = (rows0, rows1)
  wfb = (wfb0, wfb1)
  gsem = (g0, g1)
  wsem = (w0, w1)
  ssem = (s0, s1)

  @pl.loop(0, CH * (NF // 16))
  def _(t):
    r = t // (NF // 16)
    v = t % (NF // 16)
    rows0[r, pl.ds(v * 16, 16)] = jnp.zeros((16,), jnp.float32)

  @pl.loop(0, RPT // CH)
  def _(k):
    pltpu.sync_copy(rows0, agg_sp.at[pl.ds(sid * RPT + k * CH, CH)])

  plsc.subcore_barrier()

  @pl.loop(0, NP)
  def _(h):
    pltpu.sync_copy(src_hbm.at[wid, h], src_sl)
    pltpu.sync_copy(dst_hbm.at[wid, h], dst_sl)
    cbase = (wid * NP + h) * PCH

    def issue(k, b):
      pltpu.async_copy(xl_hbm.at[src_sl.at[k]], rows[b], gsem[b])
      pltpu.async_copy(wf_hbm.at[cbase + k], wfb[b], wsem[b])

    def phase(k, b):
      @pl.when(k > 0)
      def _():
        pltpu.make_async_copy(rows[1 - b], agg_sp.at[dst_sl.at[k - 1]],
                              ssem[1 - b]).wait()

      @pl.when(k + 1 < PCH)
      def _():
        issue(k + 1, 1 - b)

      pltpu.make_async_copy(xl_hbm.at[src_sl.at[k]], rows[b], gsem[b]).wait()
      pltpu.make_async_copy(wf_hbm.at[cbase + k], wfb[b], wsem[b]).wait()

      for e in range(CH):
        for v in range(NF // 16):
          sl = pl.ds(v * 16, 16)
          rows[b][e, sl] = rows[b][e, sl] * wfb[b][e, sl]

      pltpu.async_copy(rows[b], agg_sp.at[dst_sl.at[k]], ssem[b], add=True)

    issue(0, 0)

    @pl.loop(0, PCH // 2)
    def _(m):
      phase(2 * m, 0)
      phase(2 * m + 1, 1)

    pltpu.make_async_copy(rows[1], agg_sp.at[dst_sl.at[PCH - 1]],
                          ssem[1]).wait()

  plsc.subcore_barrier()
  pltpu.sync_copy(agg_sp.at[pl.ds(sid * RPT, RPT)],
                  out_hbm.at[cid, pl.ds(sid * RPT, RPT)])


def _message_pass(xl, wf, src_r, dst_r):
  return pl.kernel(
      _mp_body,
      out_type=jax.ShapeDtypeStruct((NC, NPAD, NF), jnp.float32),
      mesh=plsc.VectorSubcoreMesh(**_MESH),
      scratch_types=[
          pltpu.VMEM((PCH, CH), jnp.int32),
          pltpu.VMEM((PCH, CH), jnp.int32),
          pltpu.VMEM((CH, NF), jnp.float32),
          pltpu.VMEM((CH, NF), jnp.float32),
          pltpu.VMEM((CH, NF), jnp.float32),
          pltpu.VMEM((CH, NF), jnp.float32),
          pltpu.VMEM_SHARED((NPAD, NF), jnp.float32),
          pltpu.SemaphoreType.DMA,
          pltpu.SemaphoreType.DMA,
          pltpu.SemaphoreType.DMA,
          pltpu.SemaphoreType.DMA,
          pltpu.SemaphoreType.DMA,
          pltpu.SemaphoreType.DMA,
      ],
  )(xl, wf, src_r, dst_r)


def _node_body(has_next, h_ref, agga_ref, aggb_ref, cw2t_ref, cb2_ref,
               lwt_ref, lb_ref, cw1t_ref, hn_ref, xl_ref):
  agg = agga_ref[0] + aggb_ref[0]
  x2 = jnp.dot(agg, cw2t_ref[...], preferred_element_type=jnp.float32)
  x2 = jax.nn.gelu(x2 + cb2_ref[...])
  x2 = jnp.dot(x2, lwt_ref[...], preferred_element_type=jnp.float32)
  hn = h_ref[...] + x2 + lb_ref[...]
  hn_ref[...] = hn
  if has_next:
    xl_ref[...] = jnp.dot(hn, cw1t_ref[...],
                          preferred_element_type=jnp.float32)


def _node_update(h, agg2, c_w2t, c_b2, l_wt, l_b, c_w1t_next):
  BN = 1000
  grid = N // BN
  has_next = c_w1t_next is not None
  if not has_next:
    c_w1t_next = c_w2t
  full = lambda shape: pl.BlockSpec(shape, lambda b: tuple(0 for _ in shape))
  return pl.pallas_call(
      functools.partial(_node_body, has_next),
      grid=(grid,),
      in_specs=[
          pl.BlockSpec((BN, H), lambda b: (b, 0)),
          pl.BlockSpec((1, BN, NF), lambda b: (0, b, 0)),
          pl.BlockSpec((1, BN, NF), lambda b: (1, b, 0)),
          full((NF, H)),
          full((H,)),
          full((H, H)),
          full((H,)),
          full((H, NF)),
      ],
      out_specs=[
          pl.BlockSpec((BN, H), lambda b: (b, 0)),
          pl.BlockSpec((BN, NF), lambda b: (b, 0)),
      ],
      out_shape=[
          jax.ShapeDtypeStruct((N, H), jnp.float32),
          jax.ShapeDtypeStruct((N, NF), jnp.float32),
      ],
  )(h, agg2, agg2, c_w2t, c_b2, l_wt, l_b, c_w1t_next)


def kernel(z, edge_index, edge_length, edge_attr, emb_table,
           f_w1, f_b1, f_w2, f_b2, c_w1, c_w2, c_b2, l_w, l_b):
  z = z.astype(jnp.int32)
  src = edge_index[0].astype(jnp.int32).reshape(NW, NP, PCH, CH)
  dst_r = edge_index[1].astype(jnp.int32).reshape(NW, 1, EW)
  el_r = edge_length.reshape(NW, 1, EW)
  z_pad = jnp.concatenate([z, jnp.zeros((NPAD - N,), jnp.int32)])
  z_pad = z_pad.reshape(NW, ZCH, ICH)

  f_w1t = f_w1.transpose(0, 2, 1)
  f_w2t = f_w2.transpose(0, 2, 1)
  c_w1t = c_w1.transpose(0, 2, 1)
  c_w2t = c_w2.transpose(0, 2, 1)
  l_wt = l_w.transpose(0, 2, 1)

  wf0, wf1, wf2, xemb = _filters(edge_attr, emb_table,
                                 f_w1t, f_b1, f_w2t, f_b2, c_w1t[0])
  wfs = (wf0, wf1, wf2)

  h_pad, xl_pad, dstm = _init_gather(z_pad, emb_table, xemb, dst_r, el_r)
  dst = dstm.reshape(NW, NP, PCH, CH)
  h, xl = h_pad, xl_pad

  for i in range(L):
    agg2 = _message_pass(xl, wfs[i], src, dst)
    nxt = c_w1t[i + 1] if i + 1 < L else None
    h, xl = _node_update(h, agg2, c_w2t[i], c_b2[i], l_wt[i], l_b[i], nxt)
  return h


# speedup vs baseline: 1.6816x; 1.5708x over previous
import functools

import jax
import jax.numpy as jnp
import numpy as np
from jax import lax
from jax.experimental import pallas as pl
from jax.experimental.pallas import tpu as pltpu
from jax.experimental.pallas import tpu_sc as plsc

N = 10000
E = 320000
H = 128
NF = 128
EC = 128
L = 3
CUTOFF = 10.0

NC = 2
NS = 16
NW = NC * NS
EW = E // NW
CH = 40
NCH = EW // CH
ECH = E // CH
NP = 5
PCH = NCH // NP
NPAD = 10240
RPT = NPAD // NS
ICH = 80
ZCH = NPAD // NW // ICH

_MESH = dict(core_axis_name="c", subcore_axis_name="s", num_cores=NC,
             num_subcores=NS)


def _filter_body(ea_ref, emb_ref, fw1_ref, fb1_ref, fw2_ref, fb2_ref,
                 cw1t0_ref, wf0_ref, wf1_ref, wf2_ref, xemb_ref):
  BE = ea_ref.shape[0]
  a = ea_ref[...].astype(jnp.bfloat16)
  outs = (wf0_ref, wf1_ref, wf2_ref)
  for i in range(L):
    t = jnp.dot(a, fw1_ref[i].astype(jnp.bfloat16),
                preferred_element_type=jnp.float32)
    t = jax.nn.gelu(t + fb1_ref[i]).astype(jnp.bfloat16)
    t = jnp.dot(t, fw2_ref[i].astype(jnp.bfloat16),
                preferred_element_type=jnp.float32)
    w = (t + fb2_ref[i]).astype(jnp.bfloat16)
    wi = pltpu.bitcast(w, jnp.int32)
    outs[i][...] = wi.reshape(BE // CH, CH // 2, NF)

  @pl.when(pl.program_id(0) == 0)
  def _():
    xemb_ref[...] = jnp.dot(emb_ref[...], cw1t0_ref[...],
                            preferred_element_type=jnp.float32)


def _filters(edge_attr, emb_table, f_w1t, f_b1, f_w2t, f_b2, c_w1t0):
  BE = 1280
  grid = E // BE
  full = lambda shape: pl.BlockSpec(shape, lambda b: tuple(0 for _ in shape))
  wf_spec = pl.BlockSpec((BE // CH, CH // 2, NF), lambda b: (b, 0, 0))
  wf_shape = jax.ShapeDtypeStruct((ECH, CH // 2, NF), jnp.int32)
  return pl.pallas_call(
      _filter_body,
      grid=(grid,),
      in_specs=[
          pl.BlockSpec((BE, EC), lambda b: (b, 0)),
          full((100, H)),
          full((L, EC, NF)),
          full((L, NF)),
          full((L, NF, NF)),
          full((L, NF)),
          full((H, NF)),
      ],
      out_specs=[wf_spec, wf_spec, wf_spec, full((100, NF))],
      out_shape=[wf_shape, wf_shape, wf_shape,
                 jax.ShapeDtypeStruct((100, NF), jnp.float32)],
  )(edge_attr, emb_table, f_w1t, f_b1, f_w2t, f_b2, c_w1t0)


EWV = EW // 16


def _init_gather_body(z_hbm, emb_hbm, xemb_hbm, dst_hbm, el_hbm,
                      h0_hbm, xl0_hbm, dm_hbm,
                      zslab, hrows, xrows, dsl, esl, sem):
  wid = lax.axis_index("c") * NS + lax.axis_index("s")
  pltpu.sync_copy(z_hbm.at[wid], zslab)

  @pl.loop(0, ZCH)
  def _(j):
    pltpu.async_copy(emb_hbm.at[zslab.at[j]], hrows, sem).wait()
    pltpu.async_copy(xemb_hbm.at[zslab.at[j]], xrows, sem).wait()
    base = (wid * ZCH + j) * ICH
    pltpu.sync_copy(hrows, h0_hbm.at[pl.ds(base, ICH)])
    pltpu.sync_copy(xrows, xl0_hbm.at[pl.ds(base, ICH)])

  pltpu.sync_copy(dst_hbm.at[wid, 0], dsl)
  pltpu.sync_copy(el_hbm.at[wid, 0], esl)

  @pl.loop(0, EWV)
  def _(r):
    sl = pl.ds(r * 16, 16)
    dm = jnp.where(esl[sl] <= CUTOFF, dsl[sl], jnp.full((16,), N, jnp.int32))
    dsl[sl] = dm

  pltpu.sync_copy(dsl, dm_hbm.at[wid, 0])


def _init_gather(z_pad, emb_table, xemb, dst_r, el_r):
  return pl.kernel(
      _init_gather_body,
      out_type=[
          jax.ShapeDtypeStruct((NPAD, H), jnp.float32),
          jax.ShapeDtypeStruct((NPAD, NF), jnp.float32),
          jax.ShapeDtypeStruct((NW, 1, EW), jnp.int32),
      ],
      mesh=plsc.VectorSubcoreMesh(**_MESH),
      scratch_types=[
          pltpu.VMEM((ZCH, ICH), jnp.int32),
          pltpu.VMEM((ICH, H), jnp.float32),
          pltpu.VMEM((ICH, NF), jnp.float32),
          pltpu.VMEM((EW,), jnp.int32),
          pltpu.VMEM((EW,), jnp.float32),
          pltpu.SemaphoreType.DMA,
      ],
  )(z_pad, emb_table, xemb, dst_r, el_r)


def _mp_body(xl_hbm, wf_hbm, src_hbm, dst_hbm, out_hbm,
             src_sl, dst_sl, rows0, rows1, wfb0, wfb1, agg_sp,
             g0, g1, w0, w1, s0, s1):
  cid = lax.axis_index("c")
  sid = lax.axis_index("s")
  wid = cid * NS + sid
  rows = (rows0, rows1)
  wfb = (wfb0, wfb1)
  gsem = (g0, g1)
  wsem = (w0, w1)
  ssem = (s0, s1)

  @pl.loop(0, CH * (NF // 16))
  def _(t):
    r = t // (NF // 16)
    v = t % (NF // 16)
    rows0[r, pl.ds(v * 16, 16)] = jnp.zeros((16,), jnp.float32)

  @pl.loop(0, RPT // CH)
  def _(k):
    pltpu.sync_copy(rows0, agg_sp.at[pl.ds(sid * RPT + k * CH, CH)])

  plsc.subcore_barrier()

  @pl.loop(0, NP)
  def _(h):
    pltpu.sync_copy(src_hbm.at[wid, h], src_sl)
    pltpu.sync_copy(dst_hbm.at[wid, h], dst_sl)
    cbase = (wid * NP + h) * PCH

    def issue(k, b):
      pltpu.async_copy(xl_hbm.at[src_sl.at[k]], rows[b], gsem[b])
      pltpu.async_copy(wf_hbm.at[cbase + k], wfb[b], wsem[b])

    def phase(k, b):
      @pl.when(k > 0)
      def _():
        pltpu.make_async_copy(rows[1 - b], agg_sp.at[dst_sl.at[k - 1]],
                              ssem[1 - b]).wait()

      @pl.when(k + 1 < PCH)
      def _():
        issue(k + 1, 1 - b)

      pltpu.make_async_copy(xl_hbm.at[src_sl.at[k]], rows[b], gsem[b]).wait()
      pltpu.make_async_copy(wf_hbm.at[cbase + k], wfb[b], wsem[b]).wait()

      for p in range(CH // 2):
        for v in range(NF // 16):
          sl = pl.ds(v * 16, 16)
          w32 = wfb[b][p, sl]
          lo = plsc.bitcast(w32 << 16, jnp.float32)
          hi = plsc.bitcast(w32 & jnp.int32(-65536), jnp.float32)
          rows[b][2 * p, sl] = rows[b][2 * p, sl] * lo
          rows[b][2 * p + 1, sl] = rows[b][2 * p + 1, sl] * hi

      pltpu.async_copy(rows[b], agg_sp.at[dst_sl.at[k]], ssem[b], add=True)

    issue(0, 0)

    @pl.loop(0, PCH // 2)
    def _(m):
      phase(2 * m, 0)
      phase(2 * m + 1, 1)

    pltpu.make_async_copy(rows[1], agg_sp.at[dst_sl.at[PCH - 1]],
                          ssem[1]).wait()

  plsc.subcore_barrier()
  pltpu.sync_copy(agg_sp.at[pl.ds(sid * RPT, RPT)],
                  out_hbm.at[cid, pl.ds(sid * RPT, RPT)])


def _message_pass(xl, wf, src_r, dst_r):
  return pl.kernel(
      _mp_body,
      out_type=jax.ShapeDtypeStruct((NC, NPAD, NF), jnp.float32),
      mesh=plsc.VectorSubcoreMesh(**_MESH),
      compiler_params=pltpu.CompilerParams(needs_layout_passes=False),
      scratch_types=[
          pltpu.VMEM((PCH, CH), jnp.int32),
          pltpu.VMEM((PCH, CH), jnp.int32),
          pltpu.VMEM((CH, NF), jnp.float32),
          pltpu.VMEM((CH, NF), jnp.float32),
          pltpu.VMEM((CH // 2, NF), jnp.int32),
          pltpu.VMEM((CH // 2, NF), jnp.int32),
          pltpu.VMEM_SHARED((NPAD, NF), jnp.float32),
          pltpu.SemaphoreType.DMA,
          pltpu.SemaphoreType.DMA,
          pltpu.SemaphoreType.DMA,
          pltpu.SemaphoreType.DMA,
          pltpu.SemaphoreType.DMA,
          pltpu.SemaphoreType.DMA,
      ],
  )(xl, wf, src_r, dst_r)


def _node_body(has_next, h_ref, agga_ref, aggb_ref, cw2t_ref, cb2_ref,
               lwt_ref, lb_ref, cw1t_ref, hn_ref, xl_ref):
  agg = agga_ref[0] + aggb_ref[0]
  x2 = jnp.dot(agg, cw2t_ref[...], preferred_element_type=jnp.float32)
  x2 = jax.nn.gelu(x2 + cb2_ref[...])
  x2 = jnp.dot(x2, lwt_ref[...], preferred_element_type=jnp.float32)
  hn = h_ref[...] + x2 + lb_ref[...]
  hn_ref[...] = hn
  if has_next:
    xl_ref[...] = jnp.dot(hn, cw1t_ref[...],
                          preferred_element_type=jnp.float32)


def _node_update(h, agg2, c_w2t, c_b2, l_wt, l_b, c_w1t_next):
  BN = 1000
  grid = N // BN
  has_next = c_w1t_next is not None
  if not has_next:
    c_w1t_next = c_w2t
  full = lambda shape: pl.BlockSpec(shape, lambda b: tuple(0 for _ in shape))
  return pl.pallas_call(
      functools.partial(_node_body, has_next),
      grid=(grid,),
      in_specs=[
          pl.BlockSpec((BN, H), lambda b: (b, 0)),
          pl.BlockSpec((1, BN, NF), lambda b: (0, b, 0)),
          pl.BlockSpec((1, BN, NF), lambda b: (1, b, 0)),
          full((NF, H)),
          full((H,)),
          full((H, H)),
          full((H,)),
          full((H, NF)),
      ],
      out_specs=[
          pl.BlockSpec((BN, H), lambda b: (b, 0)),
          pl.BlockSpec((BN, NF), lambda b: (b, 0)),
      ],
      out_shape=[
          jax.ShapeDtypeStruct((N, H), jnp.float32),
          jax.ShapeDtypeStruct((N, NF), jnp.float32),
      ],
  )(h, agg2, agg2, c_w2t, c_b2, l_wt, l_b, c_w1t_next)


def kernel(z, edge_index, edge_length, edge_attr, emb_table,
           f_w1, f_b1, f_w2, f_b2, c_w1, c_w2, c_b2, l_w, l_b):
  z = z.astype(jnp.int32)
  src = edge_index[0].astype(jnp.int32).reshape(NW, NP, PCH, CH)
  dst_r = edge_index[1].astype(jnp.int32).reshape(NW, 1, EW)
  el_r = edge_length.reshape(NW, 1, EW)
  z_pad = jnp.concatenate([z, jnp.zeros((NPAD - N,), jnp.int32)])
  z_pad = z_pad.reshape(NW, ZCH, ICH)

  f_w1t = f_w1.transpose(0, 2, 1)
  f_w2t = f_w2.transpose(0, 2, 1)
  c_w1t = c_w1.transpose(0, 2, 1)
  c_w2t = c_w2.transpose(0, 2, 1)
  l_wt = l_w.transpose(0, 2, 1)

  wf0, wf1, wf2, xemb = _filters(edge_attr, emb_table,
                                 f_w1t, f_b1, f_w2t, f_b2, c_w1t[0])
  wfs = (wf0, wf1, wf2)

  h_pad, xl_pad, dstm = _init_gather(z_pad, emb_table, xemb, dst_r, el_r)
  dst = dstm.reshape(NW, NP, PCH, CH)
  h, xl = h_pad, xl_pad

  for i in range(L):
    agg2 = _message_pass(xl, wfs[i], src, dst)
    nxt = c_w1t[i + 1] if i + 1 < L else None
    h, xl = _node_update(h, agg2, c_w2t[i], c_b2[i], l_wt[i], l_b[i], nxt)
  return h


# speedup vs baseline: 1.7534x; 1.0427x over previous
import functools

import jax
import jax.numpy as jnp
import numpy as np
from jax import lax
from jax.experimental import pallas as pl
from jax.experimental.pallas import tpu as pltpu
from jax.experimental.pallas import tpu_sc as plsc

N = 10000
E = 320000
H = 128
NF = 128
EC = 128
L = 3
CUTOFF = 10.0

NC = 2
NS = 16
NW = NC * NS
EW = E // NW
CH = 40
NCH = EW // CH
ECH = E // CH
NP = 5
PCH = NCH // NP
NPAD = 10240
RPT = NPAD // NS
ICH = 80
ZCH = NPAD // NW // ICH

_MESH = dict(core_axis_name="c", subcore_axis_name="s", num_cores=NC,
             num_subcores=NS)


def _filter_body(nl, ea_ref, fw1_ref, fb1_ref, fw2_ref, fb2_ref, *wf_refs):
  BE = ea_ref.shape[0]
  a = ea_ref[...].astype(jnp.bfloat16)
  for i in range(nl):
    t = jnp.dot(a, fw1_ref[i].astype(jnp.bfloat16),
                preferred_element_type=jnp.float32)
    t = jax.nn.gelu(t + fb1_ref[i]).astype(jnp.bfloat16)
    t = jnp.dot(t, fw2_ref[i].astype(jnp.bfloat16),
                preferred_element_type=jnp.float32)
    w = (t + fb2_ref[i]).astype(jnp.bfloat16)
    wi = pltpu.bitcast(w, jnp.int32)
    wf_refs[i][...] = wi.reshape(BE // CH, CH // 2, NF)


def _filters(edge_attr, f_w1t, f_b1, f_w2t, f_b2):
  nl = f_w1t.shape[0]
  BE = 1280
  grid = E // BE
  full = lambda shape: pl.BlockSpec(shape, lambda b: tuple(0 for _ in shape))
  wf_spec = pl.BlockSpec((BE // CH, CH // 2, NF), lambda b: (b, 0, 0))
  wf_shape = jax.ShapeDtypeStruct((ECH, CH // 2, NF), jnp.int32)
  return pl.pallas_call(
      functools.partial(_filter_body, nl),
      grid=(grid,),
      in_specs=[
          pl.BlockSpec((BE, EC), lambda b: (b, 0)),
          full((nl, EC, NF)),
          full((nl, NF)),
          full((nl, NF, NF)),
          full((nl, NF)),
      ],
      out_specs=[wf_spec] * nl,
      out_shape=[wf_shape] * nl,
  )(edge_attr, f_w1t, f_b1, f_w2t, f_b2)


def _xemb_body(emb_ref, cw1t0_ref, xemb_ref):
  xemb_ref[...] = jnp.dot(emb_ref[...], cw1t0_ref[...],
                          preferred_element_type=jnp.float32)


def _xemb(emb_table, c_w1t0):
  return pl.pallas_call(
      _xemb_body,
      out_shape=jax.ShapeDtypeStruct((100, NF), jnp.float32),
  )(emb_table, c_w1t0)


EWV = EW // 16


def _init_gather_body(z_hbm, emb_hbm, xemb_hbm, dst_hbm, el_hbm,
                      h0_hbm, xl0_hbm, dm_hbm,
                      zslab, hrows, xrows, dsl, esl, sem):
  wid = lax.axis_index("c") * NS + lax.axis_index("s")
  pltpu.sync_copy(z_hbm.at[wid], zslab)

  @pl.loop(0, ZCH)
  def _(j):
    pltpu.async_copy(emb_hbm.at[zslab.at[j]], hrows, sem).wait()
    pltpu.async_copy(xemb_hbm.at[zslab.at[j]], xrows, sem).wait()
    base = (wid * ZCH + j) * ICH
    pltpu.sync_copy(hrows, h0_hbm.at[pl.ds(base, ICH)])
    pltpu.sync_copy(xrows, xl0_hbm.at[pl.ds(base, ICH)])

  pltpu.sync_copy(dst_hbm.at[wid, 0], dsl)
  pltpu.sync_copy(el_hbm.at[wid, 0], esl)

  @pl.loop(0, EWV)
  def _(r):
    sl = pl.ds(r * 16, 16)
    dm = jnp.where(esl[sl] <= CUTOFF, dsl[sl], jnp.full((16,), N, jnp.int32))
    dsl[sl] = dm

  pltpu.sync_copy(dsl, dm_hbm.at[wid, 0])


def _init_gather(z_pad, emb_table, xemb, dst_r, el_r):
  return pl.kernel(
      _init_gather_body,
      out_type=[
          jax.ShapeDtypeStruct((NPAD, H), jnp.float32),
          jax.ShapeDtypeStruct((NPAD, NF), jnp.float32),
          jax.ShapeDtypeStruct((NW, 1, EW), jnp.int32),
      ],
      mesh=plsc.VectorSubcoreMesh(**_MESH),
      scratch_types=[
          pltpu.VMEM((ZCH, ICH), jnp.int32),
          pltpu.VMEM((ICH, H), jnp.float32),
          pltpu.VMEM((ICH, NF), jnp.float32),
          pltpu.VMEM((EW,), jnp.int32),
          pltpu.VMEM((EW,), jnp.float32),
          pltpu.SemaphoreType.DMA,
      ],
  )(z_pad, emb_table, xemb, dst_r, el_r)


def _mp_body(xl_hbm, wf_hbm, src_hbm, dst_hbm, out_hbm,
             src_sl, dst_sl, rows0, rows1, wfb0, wfb1, agg_sp,
             g0, g1, w0, w1, s0, s1):
  cid = lax.axis_index("c")
  sid = lax.axis_index("s")
  wid = cid * NS + sid
  rows = (rows0, rows1)
  wfb = (wfb0, wfb1)
  gsem = (g0, g1)
  wsem = (w0, w1)
  ssem = (s0, s1)

  @pl.loop(0, CH * (NF // 16))
  def _(t):
    r = t // (NF // 16)
    v = t % (NF // 16)
    rows0[r, pl.ds(v * 16, 16)] = jnp.zeros((16,), jnp.float32)

  @pl.loop(0, RPT // CH)
  def _(k):
    pltpu.sync_copy(rows0, agg_sp.at[pl.ds(sid * RPT + k * CH, CH)])

  plsc.subcore_barrier()

  @pl.loop(0, NP)
  def _(h):
    pltpu.sync_copy(src_hbm.at[wid, h], src_sl)
    pltpu.sync_copy(dst_hbm.at[wid, h], dst_sl)
    cbase = (wid * NP + h) * PCH

    def issue(k, b):
      pltpu.async_copy(xl_hbm.at[src_sl.at[k]], rows[b], gsem[b])
      pltpu.async_copy(wf_hbm.at[cbase + k], wfb[b], wsem[b])

    def phase(k, b):
      @pl.when(k > 0)
      def _():
        pltpu.make_async_copy(rows[1 - b], agg_sp.at[dst_sl.at[k - 1]],
                              ssem[1 - b]).wait()

      @pl.when(k + 1 < PCH)
      def _():
        issue(k + 1, 1 - b)

      pltpu.make_async_copy(xl_hbm.at[src_sl.at[k]], rows[b], gsem[b]).wait()
      pltpu.make_async_copy(wf_hbm.at[cbase + k], wfb[b], wsem[b]).wait()

      for p in range(CH // 2):
        for v in range(NF // 16):
          sl = pl.ds(v * 16, 16)
          w32 = wfb[b][p, sl]
          lo = plsc.bitcast(w32 << 16, jnp.float32)
          hi = plsc.bitcast(w32 & jnp.int32(-65536), jnp.float32)
          rows[b][2 * p, sl] = rows[b][2 * p, sl] * lo
          rows[b][2 * p + 1, sl] = rows[b][2 * p + 1, sl] * hi

      pltpu.async_copy(rows[b], agg_sp.at[dst_sl.at[k]], ssem[b], add=True)

    issue(0, 0)

    @pl.loop(0, PCH // 2)
    def _(m):
      phase(2 * m, 0)
      phase(2 * m + 1, 1)

    pltpu.make_async_copy(rows[1], agg_sp.at[dst_sl.at[PCH - 1]],
                          ssem[1]).wait()

  plsc.subcore_barrier()
  pltpu.sync_copy(agg_sp.at[pl.ds(sid * RPT, RPT)],
                  out_hbm.at[cid, pl.ds(sid * RPT, RPT)])


def _message_pass(xl, wf, src_r, dst_r):
  return pl.kernel(
      _mp_body,
      out_type=jax.ShapeDtypeStruct((NC, NPAD, NF), jnp.float32),
      mesh=plsc.VectorSubcoreMesh(**_MESH),
      compiler_params=pltpu.CompilerParams(needs_layout_passes=False),
      scratch_types=[
          pltpu.VMEM((PCH, CH), jnp.int32),
          pltpu.VMEM((PCH, CH), jnp.int32),
          pltpu.VMEM((CH, NF), jnp.float32),
          pltpu.VMEM((CH, NF), jnp.float32),
          pltpu.VMEM((CH // 2, NF), jnp.int32),
          pltpu.VMEM((CH // 2, NF), jnp.int32),
          pltpu.VMEM_SHARED((NPAD, NF), jnp.float32),
          pltpu.SemaphoreType.DMA,
          pltpu.SemaphoreType.DMA,
          pltpu.SemaphoreType.DMA,
          pltpu.SemaphoreType.DMA,
          pltpu.SemaphoreType.DMA,
          pltpu.SemaphoreType.DMA,
      ],
  )(xl, wf, src_r, dst_r)


def _node_body(has_next, h_ref, agga_ref, aggb_ref, cw2t_ref, cb2_ref,
               lwt_ref, lb_ref, cw1t_ref, hn_ref, xl_ref):
  agg = agga_ref[0] + aggb_ref[0]
  x2 = jnp.dot(agg, cw2t_ref[...], preferred_element_type=jnp.float32)
  x2 = jax.nn.gelu(x2 + cb2_ref[...])
  x2 = jnp.dot(x2, lwt_ref[...], preferred_element_type=jnp.float32)
  hn = h_ref[...] + x2 + lb_ref[...]
  hn_ref[...] = hn
  if has_next:
    xl_ref[...] = jnp.dot(hn, cw1t_ref[...],
                          preferred_element_type=jnp.float32)


def _node_update(h, agg2, c_w2t, c_b2, l_wt, l_b, c_w1t_next):
  BN = 1000
  grid = N // BN
  has_next = c_w1t_next is not None
  if not has_next:
    c_w1t_next = c_w2t
  full = lambda shape: pl.BlockSpec(shape, lambda b: tuple(0 for _ in shape))
  return pl.pallas_call(
      functools.partial(_node_body, has_next),
      grid=(grid,),
      in_specs=[
          pl.BlockSpec((BN, H), lambda b: (b, 0)),
          pl.BlockSpec((1, BN, NF), lambda b: (0, b, 0)),
          pl.BlockSpec((1, BN, NF), lambda b: (1, b, 0)),
          full((NF, H)),
          full((H,)),
          full((H, H)),
          full((H,)),
          full((H, NF)),
      ],
      out_specs=[
          pl.BlockSpec((BN, H), lambda b: (b, 0)),
          pl.BlockSpec((BN, NF), lambda b: (b, 0)),
      ],
      out_shape=[
          jax.ShapeDtypeStruct((N, H), jnp.float32),
          jax.ShapeDtypeStruct((N, NF), jnp.float32),
      ],
  )(h, agg2, agg2, c_w2t, c_b2, l_wt, l_b, c_w1t_next)


def kernel(z, edge_index, edge_length, edge_attr, emb_table,
           f_w1, f_b1, f_w2, f_b2, c_w1, c_w2, c_b2, l_w, l_b):
  z = z.astype(jnp.int32)
  src = edge_index[0].astype(jnp.int32).reshape(NW, NP, PCH, CH)
  dst_r = edge_index[1].astype(jnp.int32).reshape(NW, 1, EW)
  el_r = edge_length.reshape(NW, 1, EW)
  z_pad = jnp.concatenate([z, jnp.zeros((NPAD - N,), jnp.int32)])
  z_pad = z_pad.reshape(NW, ZCH, ICH)

  f_w1t = f_w1.transpose(0, 2, 1)
  f_w2t = f_w2.transpose(0, 2, 1)
  c_w1t = c_w1.transpose(0, 2, 1)
  c_w2t = c_w2.transpose(0, 2, 1)
  l_wt = l_w.transpose(0, 2, 1)

  xemb = _xemb(emb_table, c_w1t[0])
  h_pad, xl_pad, dstm = _init_gather(z_pad, emb_table, xemb, dst_r, el_r)
  dst = dstm.reshape(NW, NP, PCH, CH)
  h, xl = h_pad, xl_pad

  (wf0,) = _filters(edge_attr, f_w1t[:1], f_b1[:1], f_w2t[:1], f_b2[:1])
  wf12 = _filters(edge_attr, f_w1t[1:], f_b1[1:], f_w2t[1:], f_b2[1:])
  wfs = (wf0,) + tuple(wf12)

  for i in range(L):
    agg2 = _message_pass(xl, wfs[i], src, dst)
    nxt = c_w1t[i + 1] if i + 1 < L else None
    h, xl = _node_update(h, agg2, c_w2t[i], c_b2[i], l_wt[i], l_b[i], nxt)
  return h


# speedup vs baseline: 1.7606x; 1.0041x over previous
import functools

import jax
import jax.numpy as jnp
import numpy as np
from jax import lax
from jax.experimental import pallas as pl
from jax.experimental.pallas import tpu as pltpu
from jax.experimental.pallas import tpu_sc as plsc

N = 10000
E = 320000
H = 128
NF = 128
EC = 128
L = 3
CUTOFF = 10.0

NC = 2
NS = 16
NW = NC * NS
EW = E // NW
CH = 40
NCH = EW // CH
ECH = E // CH
NP = 5
PCH = NCH // NP
NPAD = 10240
RPT = NPAD // NS
ICH = 80
ZCH = NPAD // NW // ICH

_MESH = dict(core_axis_name="c", subcore_axis_name="s", num_cores=NC,
             num_subcores=NS)


def _filter_body(nl, ea_ref, fw1_ref, fb1_ref, fw2_ref, fb2_ref, *wf_refs):
  BE = ea_ref.shape[0]
  a = ea_ref[...].astype(jnp.bfloat16)
  for i in range(nl):
    t = jnp.dot(a, fw1_ref[i].astype(jnp.bfloat16),
                preferred_element_type=jnp.float32)
    t = jax.nn.gelu(t + fb1_ref[i]).astype(jnp.bfloat16)
    t = jnp.dot(t, fw2_ref[i].astype(jnp.bfloat16),
                preferred_element_type=jnp.float32)
    w = (t + fb2_ref[i]).astype(jnp.bfloat16)
    wi = pltpu.bitcast(w, jnp.int32)
    wf_refs[i][...] = wi.reshape(BE // CH, CH // 2, NF)


def _filters(edge_attr, f_w1t, f_b1, f_w2t, f_b2):
  nl = f_w1t.shape[0]
  BE = 1280
  grid = E // BE
  full = lambda shape: pl.BlockSpec(shape, lambda b: tuple(0 for _ in shape))
  wf_spec = pl.BlockSpec((BE // CH, CH // 2, NF), lambda b: (b, 0, 0))
  wf_shape = jax.ShapeDtypeStruct((ECH, CH // 2, NF), jnp.int32)
  return pl.pallas_call(
      functools.partial(_filter_body, nl),
      grid=(grid,),
      in_specs=[
          pl.BlockSpec((BE, EC), lambda b: (b, 0)),
          full((nl, EC, NF)),
          full((nl, NF)),
          full((nl, NF, NF)),
          full((nl, NF)),
      ],
      out_specs=[wf_spec] * nl,
      out_shape=[wf_shape] * nl,
  )(edge_attr, f_w1t, f_b1, f_w2t, f_b2)


def _xemb_body(emb_ref, cw1t0_ref, xemb_ref):
  xemb_ref[...] = jnp.dot(emb_ref[...], cw1t0_ref[...],
                          preferred_element_type=jnp.float32)


def _xemb(emb_table, c_w1t0):
  return pl.pallas_call(
      _xemb_body,
      out_shape=jax.ShapeDtypeStruct((100, NF), jnp.float32),
  )(emb_table, c_w1t0)


EWV = EW // 16


def _init_gather_body(z_hbm, emb_hbm, xemb_hbm, dst_hbm, el_hbm,
                      h0_hbm, xl0_hbm, dm_hbm,
                      zslab, hrows, xrows, dsl, esl, sem):
  wid = lax.axis_index("c") * NS + lax.axis_index("s")
  pltpu.sync_copy(z_hbm.at[wid], zslab)

  @pl.loop(0, ZCH)
  def _(j):
    pltpu.async_copy(emb_hbm.at[zslab.at[j]], hrows, sem).wait()
    pltpu.async_copy(xemb_hbm.at[zslab.at[j]], xrows, sem).wait()
    base = (wid * ZCH + j) * ICH
    pltpu.sync_copy(hrows, h0_hbm.at[pl.ds(base, ICH)])
    pltpu.sync_copy(xrows, xl0_hbm.at[pl.ds(base, ICH)])

  pltpu.sync_copy(dst_hbm.at[wid, 0], dsl)
  pltpu.sync_copy(el_hbm.at[wid, 0], esl)

  @pl.loop(0, EWV)
  def _(r):
    sl = pl.ds(r * 16, 16)
    dm = jnp.where(esl[sl] <= CUTOFF, dsl[sl], jnp.full((16,), N, jnp.int32))
    dsl[sl] = dm

  pltpu.sync_copy(dsl, dm_hbm.at[wid, 0])


def _init_gather(z_pad, emb_table, xemb, dst_r, el_r):
  return pl.kernel(
      _init_gather_body,
      out_type=[
          jax.ShapeDtypeStruct((NPAD, H), jnp.float32),
          jax.ShapeDtypeStruct((NPAD, NF), jnp.float32),
          jax.ShapeDtypeStruct((NW, 1, EW), jnp.int32),
      ],
      mesh=plsc.VectorSubcoreMesh(**_MESH),
      scratch_types=[
          pltpu.VMEM((ZCH, ICH), jnp.int32),
          pltpu.VMEM((ICH, H), jnp.float32),
          pltpu.VMEM((ICH, NF), jnp.float32),
          pltpu.VMEM((EW,), jnp.int32),
          pltpu.VMEM((EW,), jnp.float32),
          pltpu.SemaphoreType.DMA,
      ],
  )(z_pad, emb_table, xemb, dst_r, el_r)


_M16 = -65536


def _mp_body(xl_hbm, wf_hbm, src_hbm, dst_hbm, out_hbm,
             src_sl, dst_sl, rows0, rows1, wfb0, wfb1, msg0, msg1, agg_sp,
             g0, g1, w0, w1, s0, s1):
  cid = lax.axis_index("c")
  sid = lax.axis_index("s")
  wid = cid * NS + sid
  rows = (rows0, rows1)
  wfb = (wfb0, wfb1)
  msg = (msg0, msg1)
  gsem = (g0, g1)
  wsem = (w0, w1)
  ssem = (s0, s1)

  @pl.loop(0, CH * (NF // 16))
  def _(t):
    r = t // (NF // 16)
    v = t % (NF // 16)
    msg0[r, pl.ds(v * 16, 16)] = jnp.zeros((16,), jnp.float32)

  @pl.loop(0, RPT // CH)
  def _(k):
    pltpu.sync_copy(msg0, agg_sp.at[pl.ds(sid * RPT + k * CH, CH)])

  plsc.subcore_barrier()

  def bc(x):
    return plsc.bitcast(x, jnp.float32)

  @pl.loop(0, NP)
  def _(h):
    pltpu.sync_copy(src_hbm.at[wid, h, 0], src_sl)
    pltpu.sync_copy(dst_hbm.at[wid, h], dst_sl)
    cbase = (wid * NP + h) * PCH

    def issue(k, b):
      pltpu.async_copy(xl_hbm.at[src_sl.at[pl.ds(k * CH, CH)]],
                       rows[b], gsem[b])
      pltpu.async_copy(wf_hbm.at[cbase + k], wfb[b], wsem[b])

    def phase(k, b):
      @pl.when(k > 0)
      def _():
        pltpu.make_async_copy(msg[1 - b], agg_sp.at[dst_sl.at[k - 1]],
                              ssem[1 - b]).wait()

      @pl.when(k + 1 < PCH)
      def _():
        issue(k + 1, 1 - b)

      pltpu.make_async_copy(xl_hbm.at[src_sl.at[pl.ds(k * CH, CH)]],
                            rows[b], gsem[b]).wait()
      pltpu.make_async_copy(wf_hbm.at[cbase + k], wfb[b], wsem[b]).wait()

      for p in range(CH // 2):
        for v in range(NF // 16):
          sl = pl.ds(v * 16, 16)
          w32 = wfb[b][p, sl]
          lo = bc(w32 << 16)
          hi = bc(w32 & _M16)
          msg[b][2 * p, sl] = rows[b][2 * p, sl] * lo
          msg[b][2 * p + 1, sl] = rows[b][2 * p + 1, sl] * hi

      pltpu.async_copy(msg[b], agg_sp.at[dst_sl.at[k]], ssem[b], add=True)

    issue(0, 0)

    @pl.loop(0, PCH // 2)
    def _(m):
      phase(2 * m, 0)
      phase(2 * m + 1, 1)

    pltpu.make_async_copy(msg[1], agg_sp.at[dst_sl.at[PCH - 1]],
                          ssem[1]).wait()

  plsc.subcore_barrier()
  pltpu.sync_copy(agg_sp.at[pl.ds(sid * RPT, RPT)],
                  out_hbm.at[cid, pl.ds(sid * RPT, RPT)])


def _message_pass(xlp, wf, src_r, dst_r):
  return pl.kernel(
      _mp_body,
      out_type=jax.ShapeDtypeStruct((NC, NPAD, NF), jnp.float32),
      mesh=plsc.VectorSubcoreMesh(**_MESH),
      compiler_params=pltpu.CompilerParams(needs_layout_passes=False),
      scratch_types=[
          pltpu.VMEM((PCH * CH,), jnp.int32),
          pltpu.VMEM((PCH, CH), jnp.int32),
          pltpu.VMEM((CH, NF), jnp.float32),
          pltpu.VMEM((CH, NF), jnp.float32),
          pltpu.VMEM((CH // 2, NF), jnp.int32),
          pltpu.VMEM((CH // 2, NF), jnp.int32),
          pltpu.VMEM((CH, NF), jnp.float32),
          pltpu.VMEM((CH, NF), jnp.float32),
          pltpu.VMEM_SHARED((NPAD, NF), jnp.float32),
          pltpu.SemaphoreType.DMA,
          pltpu.SemaphoreType.DMA,
          pltpu.SemaphoreType.DMA,
          pltpu.SemaphoreType.DMA,
          pltpu.SemaphoreType.DMA,
          pltpu.SemaphoreType.DMA,
      ],
  )(xlp, wf, src_r, dst_r)


def _node_body(has_next, h_ref, agga_ref, aggb_ref, cw2t_ref, cb2_ref,
               lwt_ref, lb_ref, cw1t_ref, hn_ref, xl_ref):
  agg = agga_ref[0] + aggb_ref[0]
  x2 = jnp.dot(agg, cw2t_ref[...], preferred_element_type=jnp.float32)
  x2 = jax.nn.gelu(x2 + cb2_ref[...])
  x2 = jnp.dot(x2, lwt_ref[...], preferred_element_type=jnp.float32)
  hn = h_ref[...] + x2 + lb_ref[...]
  hn_ref[...] = hn
  if has_next:
    xl_ref[...] = jnp.dot(hn, cw1t_ref[...],
                          preferred_element_type=jnp.float32)


def _node_update(h, agg2, c_w2t, c_b2, l_wt, l_b, c_w1t_next):
  BN = 1000
  grid = N // BN
  has_next = c_w1t_next is not None
  if not has_next:
    c_w1t_next = c_w2t
  full = lambda shape: pl.BlockSpec(shape, lambda b: tuple(0 for _ in shape))
  return pl.pallas_call(
      functools.partial(_node_body, has_next),
      grid=(grid,),
      in_specs=[
          pl.BlockSpec((BN, H), lambda b: (b, 0)),
          pl.BlockSpec((1, BN, NF), lambda b: (0, b, 0)),
          pl.BlockSpec((1, BN, NF), lambda b: (1, b, 0)),
          full((NF, H)),
          full((H,)),
          full((H, H)),
          full((H,)),
          full((H, NF)),
      ],
      out_specs=[
          pl.BlockSpec((BN, H), lambda b: (b, 0)),
          pl.BlockSpec((BN, NF), lambda b: (b, 0)),
      ],
      out_shape=[
          jax.ShapeDtypeStruct((N, H), jnp.float32),
          jax.ShapeDtypeStruct((N, NF), jnp.float32),
      ],
  )(h, agg2, agg2, c_w2t, c_b2, l_wt, l_b, c_w1t_next)


def kernel(z, edge_index, edge_length, edge_attr, emb_table,
           f_w1, f_b1, f_w2, f_b2, c_w1, c_w2, c_b2, l_w, l_b):
  z = z.astype(jnp.int32)
  src = edge_index[0].astype(jnp.int32).reshape(NW, NP, 1, PCH * CH)
  dst_r = edge_index[1].astype(jnp.int32).reshape(NW, 1, EW)
  el_r = edge_length.reshape(NW, 1, EW)
  z_pad = jnp.concatenate([z, jnp.zeros((NPAD - N,), jnp.int32)])
  z_pad = z_pad.reshape(NW, ZCH, ICH)

  f_w1t = f_w1.transpose(0, 2, 1)
  f_w2t = f_w2.transpose(0, 2, 1)
  c_w1t = c_w1.transpose(0, 2, 1)
  c_w2t = c_w2.transpose(0, 2, 1)
  l_wt = l_w.transpose(0, 2, 1)

  xemb = _xemb(emb_table, c_w1t[0])
  h_pad, xl_pad, dstm = _init_gather(z_pad, emb_table, xemb, dst_r, el_r)
  dst = dstm.reshape(NW, NP, PCH, CH)
  h, xl = h_pad, xl_pad

  (wf0,) = _filters(edge_attr, f_w1t[:1], f_b1[:1], f_w2t[:1], f_b2[:1])
  wfs = [wf0, None, None]

  for i in range(L):
    agg2 = _message_pass(xl, wfs[i], src, dst)
    if i == 0:
      wfs[1], wfs[2] = _filters(edge_attr, f_w1t[1:], f_b1[1:],
                                f_w2t[1:], f_b2[1:])
    nxt = c_w1t[i + 1] if i + 1 < L else None
    h, xl = _node_update(h, agg2, c_w2t[i], c_b2[i], l_wt[i], l_b[i], nxt)
  return h


# speedup vs baseline: 1.8451x; 1.0480x over previous
import functools

import jax
import jax.numpy as jnp
import numpy as np
from jax import lax
from jax.experimental import pallas as pl
from jax.experimental.pallas import tpu as pltpu
from jax.experimental.pallas import tpu_sc as plsc

N = 10000
E = 320000
H = 128
NF = 128
EC = 128
L = 3
CUTOFF = 10.0

NC = 2
NS = 16
NW = NC * NS
EW = E // NW
CH = 40
NCH = EW // CH
ECH = E // CH
NP = 5
PCH = NCH // NP
NPAD = 10240
RPT = NPAD // NS
ICH = 80
ZCH = NPAD // NW // ICH

_MESH = dict(core_axis_name="c", subcore_axis_name="s", num_cores=NC,
             num_subcores=NS)


def _filter_body(nl, ea_ref, fw1_ref, fb1_ref, fw2_ref, fb2_ref, *wf_refs):
  BE = ea_ref.shape[0]
  a = ea_ref[...].astype(jnp.bfloat16)
  for i in range(nl):
    t = jnp.dot(a, fw1_ref[i].astype(jnp.bfloat16),
                preferred_element_type=jnp.float32)
    t = jax.nn.gelu(t + fb1_ref[i]).astype(jnp.bfloat16)
    t = jnp.dot(t, fw2_ref[i].astype(jnp.bfloat16),
                preferred_element_type=jnp.float32)
    w = (t + fb2_ref[i]).astype(jnp.bfloat16)
    wi = pltpu.bitcast(w, jnp.int32)
    wf_refs[i][...] = wi.reshape(BE // CH, CH // 2, NF)


def _filters(edge_attr, f_w1t, f_b1, f_w2t, f_b2):
  nl = f_w1t.shape[0]
  BE = 1280
  grid = E // BE
  full = lambda shape: pl.BlockSpec(shape, lambda b: tuple(0 for _ in shape))
  wf_spec = pl.BlockSpec((BE // CH, CH // 2, NF), lambda b: (b, 0, 0))
  wf_shape = jax.ShapeDtypeStruct((ECH, CH // 2, NF), jnp.int32)
  return pl.pallas_call(
      functools.partial(_filter_body, nl),
      grid=(grid,),
      in_specs=[
          pl.BlockSpec((BE, EC), lambda b: (b, 0)),
          full((nl, EC, NF)),
          full((nl, NF)),
          full((nl, NF, NF)),
          full((nl, NF)),
      ],
      out_specs=[wf_spec] * nl,
      out_shape=[wf_shape] * nl,
  )(edge_attr, f_w1t, f_b1, f_w2t, f_b2)


def _xemb_body(emb_ref, cw1t0_ref, xemb_ref):
  xemb_ref[...] = jnp.dot(emb_ref[...], cw1t0_ref[...],
                          preferred_element_type=jnp.float32)


def _xemb(emb_table, c_w1t0):
  return pl.pallas_call(
      _xemb_body,
      out_shape=jax.ShapeDtypeStruct((100, NF), jnp.float32),
  )(emb_table, c_w1t0)


EWV = EW // 16


def _init_gather_body(z_hbm, emb_hbm, xemb_hbm, dst_hbm, el_hbm,
                      h0_hbm, xl0_hbm, dm_hbm,
                      zslab, hrows, xrows, dsl, esl, sem):
  wid = lax.axis_index("c") * NS + lax.axis_index("s")
  pltpu.sync_copy(z_hbm.at[wid], zslab)

  @pl.loop(0, ZCH)
  def _(j):
    pltpu.async_copy(emb_hbm.at[zslab.at[j]], hrows, sem).wait()
    pltpu.async_copy(xemb_hbm.at[zslab.at[j]], xrows, sem).wait()
    base = (wid * ZCH + j) * ICH
    pltpu.sync_copy(hrows, h0_hbm.at[pl.ds(base, ICH)])
    pltpu.sync_copy(xrows, xl0_hbm.at[pl.ds(base, ICH)])

  pltpu.sync_copy(dst_hbm.at[wid, 0], dsl)
  pltpu.sync_copy(el_hbm.at[wid, 0], esl)

  @pl.loop(0, EWV)
  def _(r):
    sl = pl.ds(r * 16, 16)
    dm = jnp.where(esl[sl] <= CUTOFF, dsl[sl], jnp.full((16,), N, jnp.int32))
    dsl[sl] = dm

  pltpu.sync_copy(dsl, dm_hbm.at[wid, 0])


def _init_gather(z_pad, emb_table, xemb, dst_r, el_r):
  return pl.kernel(
      _init_gather_body,
      out_type=[
          jax.ShapeDtypeStruct((NPAD, H), jnp.float32),
          jax.ShapeDtypeStruct((NPAD, NF), jnp.float32),
          jax.ShapeDtypeStruct((NW, 1, EW), jnp.int32),
      ],
      mesh=plsc.VectorSubcoreMesh(**_MESH),
      scratch_types=[
          pltpu.VMEM((ZCH, ICH), jnp.int32),
          pltpu.VMEM((ICH, H), jnp.float32),
          pltpu.VMEM((ICH, NF), jnp.float32),
          pltpu.VMEM((EW,), jnp.int32),
          pltpu.VMEM((EW,), jnp.float32),
          pltpu.SemaphoreType.DMA,
      ],
  )(z_pad, emb_table, xemb, dst_r, el_r)


_M16 = -65536


def _mp_body(xl_hbm, wf_hbm, src_hbm, dst_hbm, out_hbm,
             src_sl, dst_sl, rows0, rows1, wfb0, wfb1, msg0, msg1, agg_sp,
             g0, g1, w0, w1, s0, s1):
  cid = lax.axis_index("c")
  sid = lax.axis_index("s")
  wid = cid * NS + sid
  rows = (rows0, rows1)
  wfb = (wfb0, wfb1)
  msg = (msg0, msg1)
  gsem = (g0, g1)
  wsem = (w0, w1)
  ssem = (s0, s1)

  @pl.loop(0, CH * (NF // 16))
  def _(t):
    r = t // (NF // 16)
    v = t % (NF // 16)
    msg0[r, pl.ds(v * 16, 16)] = jnp.zeros((16,), jnp.float32)

  @pl.loop(0, RPT // CH)
  def _(k):
    pltpu.sync_copy(msg0, agg_sp.at[pl.ds(sid * RPT + k * CH, CH)])

  plsc.subcore_barrier()

  def bc(x):
    return plsc.bitcast(x, jnp.float32)

  @pl.loop(0, NP)
  def _(h):
    pltpu.sync_copy(src_hbm.at[wid, h, 0], src_sl)
    pltpu.sync_copy(dst_hbm.at[wid, h], dst_sl)
    cbase = (wid * NP + h) * PCH

    def issue(k, b):
      pltpu.async_copy(xl_hbm.at[src_sl.at[pl.ds(k * CH, CH)]],
                       rows[b], gsem[b])
      pltpu.async_copy(wf_hbm.at[cbase + k], wfb[b], wsem[b])

    def phase(k, b):
      @pl.when(k + 1 < PCH)
      def _():
        issue(k + 1, 1 - b)

      pltpu.make_async_copy(xl_hbm.at[src_sl.at[pl.ds(k * CH, CH)]],
                            rows[b], gsem[b]).wait()
      pltpu.make_async_copy(wf_hbm.at[cbase + k], wfb[b], wsem[b]).wait()

      @pl.when(k >= 2)
      def _():
        pltpu.make_async_copy(msg[b], agg_sp.at[dst_sl.at[k - 2]],
                              ssem[b]).wait()

      for p in range(CH // 2):
        for v in range(NF // 16):
          sl = pl.ds(v * 16, 16)
          w32 = wfb[b][p, sl]
          lo = bc(w32 << 16)
          hi = bc(w32 & _M16)
          msg[b][2 * p, sl] = rows[b][2 * p, sl] * lo
          msg[b][2 * p + 1, sl] = rows[b][2 * p + 1, sl] * hi

      pltpu.async_copy(msg[b], agg_sp.at[dst_sl.at[k]], ssem[b], add=True)

    issue(0, 0)

    @pl.loop(0, PCH // 2)
    def _(m):
      phase(2 * m, 0)
      phase(2 * m + 1, 1)

    pltpu.make_async_copy(msg[0], agg_sp.at[dst_sl.at[PCH - 2]],
                          ssem[0]).wait()
    pltpu.make_async_copy(msg[1], agg_sp.at[dst_sl.at[PCH - 1]],
                          ssem[1]).wait()

  plsc.subcore_barrier()
  pltpu.sync_copy(agg_sp.at[pl.ds(sid * RPT, RPT)],
                  out_hbm.at[cid, pl.ds(sid * RPT, RPT)])


def _message_pass(xlp, wf, src_r, dst_r):
  return pl.kernel(
      _mp_body,
      out_type=jax.ShapeDtypeStruct((NC, NPAD, NF), jnp.float32),
      mesh=plsc.VectorSubcoreMesh(**_MESH),
      compiler_params=pltpu.CompilerParams(needs_layout_passes=False),
      scratch_types=[
          pltpu.VMEM((PCH * CH,), jnp.int32),
          pltpu.VMEM((PCH, CH), jnp.int32),
          pltpu.VMEM((CH, NF), jnp.float32),
          pltpu.VMEM((CH, NF), jnp.float32),
          pltpu.VMEM((CH // 2, NF), jnp.int32),
          pltpu.VMEM((CH // 2, NF), jnp.int32),
          pltpu.VMEM((CH, NF), jnp.float32),
          pltpu.VMEM((CH, NF), jnp.float32),
          pltpu.VMEM_SHARED((NPAD, NF), jnp.float32),
          pltpu.SemaphoreType.DMA,
          pltpu.SemaphoreType.DMA,
          pltpu.SemaphoreType.DMA,
          pltpu.SemaphoreType.DMA,
          pltpu.SemaphoreType.DMA,
          pltpu.SemaphoreType.DMA,
      ],
  )(xlp, wf, src_r, dst_r)


def _node_body(has_next, h_ref, agga_ref, aggb_ref, cw2t_ref, cb2_ref,
               lwt_ref, lb_ref, cw1t_ref, hn_ref, xl_ref):
  agg = agga_ref[0] + aggb_ref[0]
  x2 = jnp.dot(agg, cw2t_ref[...], preferred_element_type=jnp.float32)
  x2 = jax.nn.gelu(x2 + cb2_ref[...])
  x2 = jnp.dot(x2, lwt_ref[...], preferred_element_type=jnp.float32)
  hn = h_ref[...] + x2 + lb_ref[...]
  hn_ref[...] = hn
  if has_next:
    xl_ref[...] = jnp.dot(hn, cw1t_ref[...],
                          preferred_element_type=jnp.float32)


def _node_update(h, agg2, c_w2t, c_b2, l_wt, l_b, c_w1t_next):
  BN = 1000
  grid = N // BN
  has_next = c_w1t_next is not None
  if not has_next:
    c_w1t_next = c_w2t
  full = lambda shape: pl.BlockSpec(shape, lambda b: tuple(0 for _ in shape))
  return pl.pallas_call(
      functools.partial(_node_body, has_next),
      grid=(grid,),
      in_specs=[
          pl.BlockSpec((BN, H), lambda b: (b, 0)),
          pl.BlockSpec((1, BN, NF), lambda b: (0, b, 0)),
          pl.BlockSpec((1, BN, NF), lambda b: (1, b, 0)),
          full((NF, H)),
          full((H,)),
          full((H, H)),
          full((H,)),
          full((H, NF)),
      ],
      out_specs=[
          pl.BlockSpec((BN, H), lambda b: (b, 0)),
          pl.BlockSpec((BN, NF), lambda b: (b, 0)),
      ],
      out_shape=[
          jax.ShapeDtypeStruct((N, H), jnp.float32),
          jax.ShapeDtypeStruct((N, NF), jnp.float32),
      ],
  )(h, agg2, agg2, c_w2t, c_b2, l_wt, l_b, c_w1t_next)


def kernel(z, edge_index, edge_length, edge_attr, emb_table,
           f_w1, f_b1, f_w2, f_b2, c_w1, c_w2, c_b2, l_w, l_b):
  z = z.astype(jnp.int32)
  src = edge_index[0].astype(jnp.int32).reshape(NW, NP, 1, PCH * CH)
  dst_r = edge_index[1].astype(jnp.int32).reshape(NW, 1, EW)
  el_r = edge_length.reshape(NW, 1, EW)
  z_pad = jnp.concatenate([z, jnp.zeros((NPAD - N,), jnp.int32)])
  z_pad = z_pad.reshape(NW, ZCH, ICH)

  f_w1t = f_w1.transpose(0, 2, 1)
  f_w2t = f_w2.transpose(0, 2, 1)
  c_w1t = c_w1.transpose(0, 2, 1)
  c_w2t = c_w2.transpose(0, 2, 1)
  l_wt = l_w.transpose(0, 2, 1)

  xemb = _xemb(emb_table, c_w1t[0])
  h_pad, xl_pad, dstm = _init_gather(z_pad, emb_table, xemb, dst_r, el_r)
  dst = dstm.reshape(NW, NP, PCH, CH)
  h, xl = h_pad, xl_pad

  (wf0,) = _filters(edge_attr, f_w1t[:1], f_b1[:1], f_w2t[:1], f_b2[:1])
  wfs = [wf0, None, None]

  for i in range(L):
    agg2 = _message_pass(xl, wfs[i], src, dst)
    if i == 0:
      wfs[1], wfs[2] = _filters(edge_attr, f_w1t[1:], f_b1[1:],
                                f_w2t[1:], f_b2[1:])
    nxt = c_w1t[i + 1] if i + 1 < L else None
    h, xl = _node_update(h, agg2, c_w2t[i], c_b2[i], l_wt[i], l_b[i], nxt)
  return h
